# baseline (speedup 1.0000x reference)
import jax
import jax.numpy as jnp
from jax import lax
from jax.experimental import pallas as pl

B, N, C = 8, 16384, 80
BN = 2048


def _body(wm_ref, x_ref, z_ref, m_ref, oh_ref, o_ref):
    x = x_ref[0]
    z = z_ref[0]
    m = m_ref[0]
    oh = oh_ref[0]
    wm = wm_ref[...]

    swf = lax.dot_general(oh, wm, (((1,), (0,)), ((), ())),
                          preferred_element_type=jnp.float32)
    sw = swf[:, :C]

    e = jnp.exp(-jnp.abs(x))
    l1p = jnp.log1p(e)
    r = 1.0 / (1.0 + e)
    sig = jnp.where(x >= 0.0, r, 1.0 - r)
    sp = jnp.maximum(x, 0.0) + l1p
    neg = sp * sig * sig
    d = z - sig
    pos = (sp - x * z) * d * d
    o_ref[0] = sw * jnp.where(m, pos, neg)


def kernel(pred_score, gt_score, gt_target_pos_mask, labels_one_hot, weights):
    wm = jnp.tile(weights[:, None], (1, 128))

    grid = (B, N // BN)
    blk = pl.BlockSpec((1, BN, C), lambda b, i: (b, i, 0))
    out = pl.pallas_call(
        _body,
        grid=grid,
        in_specs=[
            pl.BlockSpec((C, 128), lambda b, i: (0, 0)),
            blk,
            blk,
            blk,
            blk,
        ],
        out_specs=blk,
        out_shape=jax.ShapeDtypeStruct((B, N, C), jnp.float32),
    )(wm, pred_score, gt_score, gt_target_pos_mask, labels_one_hot)
    return out


# speedup vs baseline: 3.3713x; 3.3713x over previous
import jax
import jax.numpy as jnp
from jax import lax
from jax.experimental import pallas as pl

B, N, C = 8, 16384, 80
BN = 2048


def _body(wm_ref, x_ref, z_ref, m_ref, oh_ref, o_ref):
    x = x_ref[0]
    z = z_ref[0]
    m = m_ref[0]
    oh = oh_ref[0]
    w = wm_ref[...][:, 0:1]

    sw = jnp.sum(oh * w, axis=0, keepdims=True)

    e = jnp.exp(-jnp.abs(x))
    l1p = jnp.log1p(e)
    r = 1.0 / (1.0 + e)
    sig = jnp.where(x >= 0.0, r, 1.0 - r)
    sp = jnp.maximum(x, 0.0) + l1p
    neg = sp * sig * sig
    d = z - sig
    pos = (sp - x * z) * d * d
    o_ref[0] = sw * jnp.where(m != 0, pos, neg)


def kernel(pred_score, gt_score, gt_target_pos_mask, labels_one_hot, weights):
    xT = jnp.transpose(pred_score, (0, 2, 1))
    zT = jnp.transpose(gt_score, (0, 2, 1))
    mT = jnp.transpose(gt_target_pos_mask.view(jnp.int8), (0, 2, 1))
    ohT = jnp.transpose(labels_one_hot, (0, 2, 1))
    wm = jnp.tile(weights[:, None], (1, 128))

    grid = (B, N // BN)
    blk = pl.BlockSpec((1, C, BN), lambda b, i: (b, 0, i))
    outT = pl.pallas_call(
        _body,
        grid=grid,
        in_specs=[
            pl.BlockSpec((C, 128), lambda b, i: (0, 0)),
            blk,
            blk,
            blk,
            blk,
        ],
        out_specs=blk,
        out_shape=jax.ShapeDtypeStruct((B, C, N), jnp.float32),
    )(wm, xT, zT, mT, ohT)
    return jnp.transpose(outT, (0, 2, 1))


# speedup vs baseline: 4.9612x; 1.4716x over previous
import jax
import jax.numpy as jnp
from jax import lax
from jax.experimental import pallas as pl

B, N, C = 8, 16384, 80
BN = 4096


def _body(wm_ref, x_ref, z_ref, m_ref, oh_ref, o_ref):
    x = x_ref[0]
    z = z_ref[0]
    m = m_ref[0]
    oh = oh_ref[0]
    wm = wm_ref[...]

    swf = lax.dot_general(wm, oh, (((0,), (0,)), ((), ())),
                          preferred_element_type=jnp.float32)
    sw = swf[0:C, :]

    e = jnp.exp(-jnp.abs(x))
    t = 1.0 + e
    l1p = jnp.log(t)
    r = 1.0 / t
    sig = jnp.where(x >= 0.0, r, 1.0 - r)
    sp = jnp.maximum(x, 0.0) + l1p
    neg = sp * sig * sig
    d = z - sig
    pos = (sp - x * z) * d * d
    o_ref[0] = sw * jnp.where(m != 0, pos, neg)


def kernel(pred_score, gt_score, gt_target_pos_mask, labels_one_hot, weights):
    xT = jnp.transpose(pred_score, (0, 2, 1))
    zT = jnp.transpose(gt_score, (0, 2, 1))
    mT = jnp.transpose(gt_target_pos_mask.view(jnp.int8), (0, 2, 1))
    ohT = jnp.transpose(labels_one_hot, (0, 2, 1))
    wm = jnp.tile(weights[:, None], (1, 128))

    grid = (B, N // BN)
    blk = pl.BlockSpec((1, C, BN), lambda b, i: (b, 0, i))
    outT = pl.pallas_call(
        _body,
        grid=grid,
        in_specs=[
            pl.BlockSpec((C, 128), lambda b, i: (0, 0)),
            blk,
            blk,
            blk,
            blk,
        ],
        out_specs=blk,
        out_shape=jax.ShapeDtypeStruct((B, C, N), jnp.float32),
    )(wm, xT, zT, mT, ohT)
    return jnp.transpose(outT, (0, 2, 1))


# speedup vs baseline: 6.1879x; 1.2473x over previous
import jax
import jax.numpy as jnp
from jax import lax
from jax.experimental import pallas as pl
from jax.experimental.pallas import tpu as pltpu

B, N, C = 8, 16384, 80
BN = 8192


def _body(wm_ref, x_ref, z_ref, m_ref, oh_ref, o_ref):
    x = x_ref[0]
    z = z_ref[0]
    m = m_ref[0]
    oh = oh_ref[0]
    wm = wm_ref[...]

    swf = lax.dot_general(wm, oh, (((0,), (0,)), ((), ())),
                          preferred_element_type=jnp.float32)
    sw = swf[0:C, :]

    e = jnp.exp(-jnp.abs(x))
    t = 1.0 + e
    l1p = jnp.log(t)
    r = 1.0 / t
    sig = jnp.where(x >= 0.0, r, 1.0 - r)
    sp = jnp.maximum(x, 0.0) + l1p
    neg = sp * sig * sig
    d = z - sig
    pos = (sp - x * z) * d * d
    o_ref[0] = sw * jnp.where(m != 0, pos, neg)


def kernel(pred_score, gt_score, gt_target_pos_mask, labels_one_hot, weights):
    xT = jnp.transpose(pred_score, (0, 2, 1))
    zT = jnp.transpose(gt_score, (0, 2, 1))
    mT = jnp.transpose(gt_target_pos_mask.view(jnp.int8), (0, 2, 1))
    ohT = jnp.transpose(labels_one_hot, (0, 2, 1))
    wm = jnp.tile(weights[:, None], (1, 128))

    grid = (B, N // BN)
    blk = pl.BlockSpec((1, C, BN), lambda b, i: (b, 0, i))
    outT = pl.pallas_call(
        _body,
        grid=grid,
        in_specs=[
            pl.BlockSpec((C, 128), lambda b, i: (0, 0)),
            blk,
            blk,
            blk,
            blk,
        ],
        out_specs=blk,
        out_shape=jax.ShapeDtypeStruct((B, C, N), jnp.float32),
        compiler_params=pltpu.CompilerParams(
            dimension_semantics=("parallel", "parallel"),
            allow_input_fusion=[False, False, False, True, False],
        ),
    )(wm, xT, zT, mT, ohT)
    return jnp.transpose(outT, (0, 2, 1))
